# Initial kernel scaffold; baseline (speedup 1.0000x reference)
import functools

import jax
import jax.numpy as jnp
from jax import lax
from jax.experimental import pallas as pl
from jax.experimental.pallas import tpu as pltpu
from jax.experimental.pallas import tpu_sc as plsc

N = 10000
NP = 10240
E = 320000
ER = 2560
EP = ER * 128
D = 128
H = 16
NC = 2
NS = 16
NW = NC * NS
RPW = ER // NW
NB = 16
NCHUNK = RPW // NB
NPER = NP // NS

_mesh = plsc.VectorSubcoreMesh(core_axis_name="c", subcore_axis_name="s")


@functools.partial(
    pl.kernel,
    out_type=jax.ShapeDtypeStruct((NC, NP, H), jnp.float32),
    mesh=_mesh,
    scratch_types=[
        pltpu.VMEM_SHARED((NP, H), jnp.float32),
        pltpu.VMEM_SHARED((NP, H), jnp.float32),
        pltpu.VMEM((NB, 128), jnp.int32),
        pltpu.VMEM((NB, 128), jnp.int32),
        pltpu.VMEM((128, H), jnp.float32),
        pltpu.VMEM((128, H), jnp.float32),
    ],
)
def _sc_messages(h_hbm, src_hbm, dst_hbm, aggp_hbm,
                 h_sh, agg_sh, sidx_v, didx_v, rows_v, zblk_v):
    c = lax.axis_index("c")
    s = lax.axis_index("s")

    @pl.loop(0, 128)
    def _(j):
        zblk_v[j, :] = jnp.zeros((H,), jnp.float32)

    pltpu.sync_copy(h_hbm.at[pl.ds(s * NPER, NPER)],
                    h_sh.at[pl.ds(s * NPER, NPER)])

    @pl.loop(0, NPER // 128)
    def _(k):
        pltpu.sync_copy(zblk_v, agg_sh.at[pl.ds(s * NPER + k * 128, 128)])

    plsc.subcore_barrier()

    row0 = (c * NS + s) * RPW

    @pl.loop(0, NCHUNK)
    def _(chunk):
        r = row0 + chunk * NB
        pltpu.sync_copy(src_hbm.at[pl.ds(r, NB)], sidx_v)
        pltpu.sync_copy(dst_hbm.at[pl.ds(r, NB)], didx_v)

        @pl.loop(0, NB)
        def _(j):
            pltpu.sync_copy(h_sh.at[sidx_v.at[j]], rows_v)
            pltpu.sync_copy(rows_v, agg_sh.at[didx_v.at[j]], add=True)

    plsc.subcore_barrier()
    pltpu.sync_copy(agg_sh.at[pl.ds(s * NPER, NPER)],
                    aggp_hbm.at[c, pl.ds(s * NPER, NPER)])


def _tc_prep1_body(f_ref, w1_ref, dop_ref, dip_ref,
                   h1s_ref, no_ref, ni_ref):
    deg_o = dop_ref[0, :, 0:1] + dop_ref[1, :, 0:1]
    deg_i = dip_ref[0, :, 0:1] + dip_ref[1, :, 0:1]
    no = lax.rsqrt(jnp.maximum(deg_o, 1.0))
    ni = lax.rsqrt(jnp.maximum(deg_i, 1.0))
    h = jnp.dot(f_ref[...], w1_ref[...], preferred_element_type=jnp.float32)
    h1s_ref[...] = h * no
    no_ref[...] = no
    ni_ref[...] = ni


_tc_prep1 = pl.pallas_call(
    _tc_prep1_body,
    out_shape=[
        jax.ShapeDtypeStruct((NP, H), jnp.float32),
        jax.ShapeDtypeStruct((NP, 1), jnp.float32),
        jax.ShapeDtypeStruct((NP, 1), jnp.float32),
    ],
)


def _tc_mid_body(aggp_ref, ni_ref, b1_ref, w2_ref, no_ref, h2s_ref):
    agg = aggp_ref[0] + aggp_ref[1]
    y = jnp.maximum(agg * ni_ref[...] + b1_ref[...], 0.0)
    h2 = jnp.dot(y, w2_ref[...], preferred_element_type=jnp.float32)
    h2s_ref[...] = h2 * no_ref[...]


_tc_mid = pl.pallas_call(
    _tc_mid_body,
    out_shape=jax.ShapeDtypeStruct((NP, H), jnp.float32),
)


def _tc_final_body(aggp_ref, ni_ref, b2_ref, out_ref):
    agg = aggp_ref[0] + aggp_ref[1]
    out_ref[...] = agg * ni_ref[...] + b2_ref[...]


_tc_final = pl.pallas_call(
    _tc_final_body,
    out_shape=jax.ShapeDtypeStruct((NP, H), jnp.float32),
)


def kernel(features, edge_index, W1, b1, W2, b2):
    src = edge_index[0]
    dst = edge_index[1]
    pad = jnp.full((EP - E,), N, jnp.int32)
    src_p = jnp.concatenate([src, pad]).reshape(ER, 128)
    dst_p = jnp.concatenate([dst, pad]).reshape(ER, 128)
    f_p = jnp.zeros((NP, D), jnp.float32).at[:N].set(features)

    ones_t = jnp.ones((NP, H), jnp.float32)
    dip = _sc_messages(ones_t, src_p, dst_p)
    dop = _sc_messages(ones_t, dst_p, src_p)

    h1s, no, ni = _tc_prep1(f_p, W1, dop, dip)
    agg1p = _sc_messages(h1s, src_p, dst_p)
    h2s = _tc_mid(agg1p, ni, b1.reshape(1, H), W2, no)
    agg2p = _sc_messages(h2s, src_p, dst_p)
    out_p = _tc_final(agg2p, ni, b2.reshape(1, H))
    return out_p[:N]


# speedup vs baseline: 14.1230x; 14.1230x over previous
import functools

import jax
import jax.numpy as jnp
from jax import lax
from jax.experimental import pallas as pl
from jax.experimental.pallas import tpu as pltpu
from jax.experimental.pallas import tpu_sc as plsc

N = 10000
NP = 10240
E = 320000
ER = 2560
EP = ER * 128
D = 128
H = 16
NC = 2
NS = 16
NW = NC * NS
RPW = ER // NW
NB = 16
NCHUNK = RPW // NB
NPER = NP // NS

_mesh = plsc.VectorSubcoreMesh(core_axis_name="c", subcore_axis_name="s")
_sc_params = pltpu.CompilerParams(use_tc_tiling_on_sc=False)


@functools.partial(
    pl.kernel,
    out_type=jax.ShapeDtypeStruct((NC, NP, H), jnp.float32),
    mesh=_mesh,
    compiler_params=_sc_params,
    scratch_types=[
        pltpu.VMEM_SHARED((NP, H), jnp.float32),
        pltpu.VMEM_SHARED((NP, H), jnp.float32),
        pltpu.VMEM((NB, 128), jnp.int32),
        pltpu.VMEM((NB, 128), jnp.int32),
        pltpu.VMEM((128, H), jnp.float32),
        pltpu.VMEM((128, H), jnp.float32),
    ],
)
def _sc_messages(h_hbm, src_hbm, dst_hbm, aggp_hbm,
                 h_sh, agg_sh, sidx_v, didx_v, rows_v, zblk_v):
    c = lax.axis_index("c")
    s = lax.axis_index("s")

    @pl.loop(0, 128)
    def _(j):
        zblk_v[j, :] = jnp.zeros((H,), jnp.float32)

    pltpu.sync_copy(h_hbm.at[pl.ds(s * NPER, NPER)],
                    h_sh.at[pl.ds(s * NPER, NPER)])

    @pl.loop(0, NPER // 128)
    def _(k):
        pltpu.sync_copy(zblk_v, agg_sh.at[pl.ds(s * NPER + k * 128, 128)])

    plsc.subcore_barrier()

    row0 = (c * NS + s) * RPW

    @pl.loop(0, NCHUNK)
    def _(chunk):
        r = row0 + chunk * NB
        pltpu.sync_copy(src_hbm.at[pl.ds(r, NB)], sidx_v)
        pltpu.sync_copy(dst_hbm.at[pl.ds(r, NB)], didx_v)

        @pl.loop(0, NB)
        def _(j):
            pltpu.sync_copy(h_sh.at[sidx_v.at[j]], rows_v)
            pltpu.sync_copy(rows_v, agg_sh.at[didx_v.at[j]], add=True)

    plsc.subcore_barrier()
    pltpu.sync_copy(agg_sh.at[pl.ds(s * NPER, NPER)],
                    aggp_hbm.at[c, pl.ds(s * NPER, NPER)])


def _tc_prep1_body(f_ref, w1_ref, dop_ref, dip_ref,
                   h1s_ref, no_ref, ni_ref):
    deg_o = dop_ref[0, :, 0:1] + dop_ref[1, :, 0:1]
    deg_i = dip_ref[0, :, 0:1] + dip_ref[1, :, 0:1]
    no = lax.rsqrt(jnp.maximum(deg_o, 1.0))
    ni = lax.rsqrt(jnp.maximum(deg_i, 1.0))
    h = jnp.dot(f_ref[...], w1_ref[...], preferred_element_type=jnp.float32)
    h1s_ref[...] = h * no
    no_ref[...] = no
    ni_ref[...] = ni


_tc_prep1 = pl.pallas_call(
    _tc_prep1_body,
    out_shape=[
        jax.ShapeDtypeStruct((NP, H), jnp.float32),
        jax.ShapeDtypeStruct((NP, 1), jnp.float32),
        jax.ShapeDtypeStruct((NP, 1), jnp.float32),
    ],
)


def _tc_mid_body(aggp_ref, ni_ref, b1_ref, w2_ref, no_ref, h2s_ref):
    agg = aggp_ref[0] + aggp_ref[1]
    y = jnp.maximum(agg * ni_ref[...] + b1_ref[...], 0.0)
    h2 = jnp.dot(y, w2_ref[...], preferred_element_type=jnp.float32)
    h2s_ref[...] = h2 * no_ref[...]


_tc_mid = pl.pallas_call(
    _tc_mid_body,
    out_shape=jax.ShapeDtypeStruct((NP, H), jnp.float32),
)


def _tc_final_body(aggp_ref, ni_ref, b2_ref, out_ref):
    agg = aggp_ref[0] + aggp_ref[1]
    out_ref[...] = agg * ni_ref[...] + b2_ref[...]


_tc_final = pl.pallas_call(
    _tc_final_body,
    out_shape=jax.ShapeDtypeStruct((NP, H), jnp.float32),
)


def kernel(features, edge_index, W1, b1, W2, b2):
    src = edge_index[0]
    dst = edge_index[1]
    pad = jnp.full((EP - E,), N, jnp.int32)
    src_p = jnp.concatenate([src, pad]).reshape(ER, 128)
    dst_p = jnp.concatenate([dst, pad]).reshape(ER, 128)
    f_p = jnp.zeros((NP, D), jnp.float32).at[:N].set(features)

    ones_t = jnp.ones((NP, H), jnp.float32)
    dip = _sc_messages(ones_t, src_p, dst_p)
    dop = _sc_messages(ones_t, dst_p, src_p)

    h1s, no, ni = _tc_prep1(f_p, W1, dop, dip)
    agg1p = _sc_messages(h1s, src_p, dst_p)
    h2s = _tc_mid(agg1p, ni, b1.reshape(1, H), W2, no)
    agg2p = _sc_messages(h2s, src_p, dst_p)
    out_p = _tc_final(agg2p, ni, b2.reshape(1, H))
    return out_p[:N]


# speedup vs baseline: 21.3275x; 1.5101x over previous
import functools

import jax
import jax.numpy as jnp
from jax import lax
from jax.experimental import pallas as pl
from jax.experimental.pallas import tpu as pltpu
from jax.experimental.pallas import tpu_sc as plsc

N = 10000
NP = 10240
E = 320000
EP = 327680
D = 128
H = 16
NC = 2
NS = 16
NW = NC * NS
EPW = EP // NW
MB = 2560
MCHUNK = EPW // MB
EPS = EP // NS
DB = 2048
DCHUNK = EPS // DB
NPER = NP // NS

_mesh = plsc.VectorSubcoreMesh(core_axis_name="c", subcore_axis_name="s")
_sc_params = pltpu.CompilerParams(use_tc_tiling_on_sc=False)


@functools.partial(
    pl.kernel,
    out_type=jax.ShapeDtypeStruct((NC, NP), jnp.float32),
    mesh=_mesh,
    compiler_params=_sc_params,
    scratch_types=[
        pltpu.VMEM_SHARED((NP,), jnp.float32),
        pltpu.VMEM((DB,), jnp.int32),
        pltpu.VMEM((DB,), jnp.float32),
        pltpu.VMEM((NPER,), jnp.float32),
    ],
)
def _sc_degrees(src_hbm, dst_hbm, deg_hbm, deg_sh, idx_v, ones_v, z_v):
    c = lax.axis_index("c")
    s = lax.axis_index("s")

    @pl.loop(0, DB // 16)
    def _(j):
        ones_v[pl.ds(j * 16, 16)] = jnp.ones((16,), jnp.float32)

    @pl.loop(0, NPER // 16)
    def _(j):
        z_v[pl.ds(j * 16, 16)] = jnp.zeros((16,), jnp.float32)

    pltpu.sync_copy(z_v, deg_sh.at[pl.ds(s * NPER, NPER)])
    plsc.subcore_barrier()

    @pl.when(c == 0)
    def _():
        @pl.loop(0, DCHUNK)
        def _(k):
            pltpu.sync_copy(src_hbm.at[pl.ds(s * EPS + k * DB, DB)], idx_v)
            pltpu.sync_copy(ones_v, deg_sh.at[idx_v], add=True)

    @pl.when(c == 1)
    def _():
        @pl.loop(0, DCHUNK)
        def _(k):
            pltpu.sync_copy(dst_hbm.at[pl.ds(s * EPS + k * DB, DB)], idx_v)
            pltpu.sync_copy(ones_v, deg_sh.at[idx_v], add=True)

    plsc.subcore_barrier()
    pltpu.sync_copy(deg_sh.at[pl.ds(s * NPER, NPER)],
                    deg_hbm.at[c, pl.ds(s * NPER, NPER)])


@functools.partial(
    pl.kernel,
    out_type=jax.ShapeDtypeStruct((NC, NP, H), jnp.float32),
    mesh=_mesh,
    compiler_params=_sc_params,
    scratch_types=[
        pltpu.VMEM_SHARED((NP, H), jnp.float32),
        pltpu.VMEM_SHARED((NP, H), jnp.float32),
        pltpu.VMEM((MB,), jnp.int32),
        pltpu.VMEM((MB,), jnp.int32),
        pltpu.VMEM((MB, H), jnp.float32),
        pltpu.VMEM((NPER, H), jnp.float32),
    ],
)
def _sc_messages(h_hbm, src_hbm, dst_hbm, aggp_hbm,
                 h_sh, agg_sh, sidx_v, didx_v, rows_v, zblk_v):
    c = lax.axis_index("c")
    s = lax.axis_index("s")

    @pl.loop(0, NPER)
    def _(j):
        zblk_v[j, :] = jnp.zeros((H,), jnp.float32)

    pltpu.sync_copy(h_hbm.at[pl.ds(s * NPER, NPER)],
                    h_sh.at[pl.ds(s * NPER, NPER)])
    pltpu.sync_copy(zblk_v, agg_sh.at[pl.ds(s * NPER, NPER)])
    plsc.subcore_barrier()

    e0 = (c * NS + s) * EPW

    @pl.loop(0, MCHUNK)
    def _(k):
        r = e0 + k * MB
        pltpu.sync_copy(src_hbm.at[pl.ds(r, MB)], sidx_v)
        pltpu.sync_copy(dst_hbm.at[pl.ds(r, MB)], didx_v)
        pltpu.sync_copy(h_sh.at[sidx_v], rows_v)
        pltpu.sync_copy(rows_v, agg_sh.at[didx_v], add=True)

    plsc.subcore_barrier()
    pltpu.sync_copy(agg_sh.at[pl.ds(s * NPER, NPER)],
                    aggp_hbm.at[c, pl.ds(s * NPER, NPER)])


def _tc_prep1_body(f_ref, w1_ref, deg_ref, h1s_ref, no_ref, ni_ref):
    no = lax.rsqrt(jnp.maximum(deg_ref[0], 1.0)).reshape(NP, 1)
    ni = lax.rsqrt(jnp.maximum(deg_ref[1], 1.0)).reshape(NP, 1)
    h = jnp.dot(f_ref[...], w1_ref[...], preferred_element_type=jnp.float32)
    h1s_ref[...] = h * no
    no_ref[...] = no
    ni_ref[...] = ni


_tc_prep1 = pl.pallas_call(
    _tc_prep1_body,
    out_shape=[
        jax.ShapeDtypeStruct((NP, H), jnp.float32),
        jax.ShapeDtypeStruct((NP, 1), jnp.float32),
        jax.ShapeDtypeStruct((NP, 1), jnp.float32),
    ],
)


def _tc_mid_body(aggp_ref, ni_ref, b1_ref, w2_ref, no_ref, h2s_ref):
    agg = aggp_ref[0] + aggp_ref[1]
    y = jnp.maximum(agg * ni_ref[...] + b1_ref[...], 0.0)
    h2 = jnp.dot(y, w2_ref[...], preferred_element_type=jnp.float32)
    h2s_ref[...] = h2 * no_ref[...]


_tc_mid = pl.pallas_call(
    _tc_mid_body,
    out_shape=jax.ShapeDtypeStruct((NP, H), jnp.float32),
)


def _tc_final_body(aggp_ref, ni_ref, b2_ref, out_ref):
    agg = aggp_ref[0] + aggp_ref[1]
    out_ref[...] = agg * ni_ref[...] + b2_ref[...]


_tc_final = pl.pallas_call(
    _tc_final_body,
    out_shape=jax.ShapeDtypeStruct((NP, H), jnp.float32),
)


def kernel(features, edge_index, W1, b1, W2, b2):
    src = edge_index[0]
    dst = edge_index[1]
    pad = jnp.full((EP - E,), N, jnp.int32)
    src_p = jnp.concatenate([src, pad])
    dst_p = jnp.concatenate([dst, pad])
    f_p = jnp.zeros((NP, D), jnp.float32).at[:N].set(features)

    deg = _sc_degrees(src_p, dst_p)
    h1s, no, ni = _tc_prep1(f_p, W1, deg)
    agg1p = _sc_messages(h1s, src_p, dst_p)
    h2s = _tc_mid(agg1p, ni, b1.reshape(1, H), W2, no)
    agg2p = _sc_messages(h2s, src_p, dst_p)
    out_p = _tc_final(agg2p, ni, b2.reshape(1, H))
    return out_p[:N]
